# Initial kernel scaffold; baseline (speedup 1.0000x reference)
import jax
import jax.numpy as jnp
from jax.experimental import pallas as pl


def kernel(X, vertex, edges, X0, W1_w, W1_b, W2_w, W2_b, W3_w1, W3_b1, W3_w2, W3_b2):
    raise NotImplementedError("write your pallas kernel here")


# speedup vs baseline: 5.9704x; 5.9704x over previous
import functools

import jax
import jax.numpy as jnp
from jax import lax
from jax.experimental import pallas as pl
from jax.experimental.pallas import tpu as pltpu
from jax.experimental.pallas import tpu_sc as plsc

NC = 2
NS = 16
LANES = 16
CH = 128

_DOT = dict(preferred_element_type=jnp.float32, precision=lax.Precision.HIGHEST)


def _tc_pre_body(x_ref, w1p_ref, b1p_ref, w2a_ref, a_ref, bp_ref):
    x = x_ref[...]
    a_ref[...] = jnp.dot(x, w1p_ref[...], **_DOT) + b1p_ref[...]
    bp_ref[...] = jnp.dot(x, w2a_ref[...], **_DOT)


def _tc_mid_body(d, se_ref, w2bp_ref, w2cp_ref, b2p_ref, c_ref):
    se = se_ref[0] + se_ref[1]
    de = jnp.sum(se[:, d:], axis=1)
    xe = se[:, :d] / jnp.maximum(de, 1.0)[:, None]
    c_ref[...] = (
        jnp.dot(xe, w2bp_ref[...], **_DOT)
        + jnp.log(de)[:, None] * w2cp_ref[...]
        + b2p_ref[...]
    )


def _tc_post_body(d, t_ref, bp_ref, x_ref, x0_ref, w3a_ref, w3b_ref,
                  w3c_ref, w3d_ref, b31_ref, w32_ref, b32_ref, out_ref):
    t = t_ref[0] + t_ref[1]
    dv = jnp.sum(t[:, d:], axis=1)
    bp = bp_ref[...]
    xv = (dv[:, None] * bp + t[:, :d]) / jnp.maximum(dv, 1.0)[:, None]
    pre = (
        jnp.dot(xv, w3a_ref[...], **_DOT)
        + jnp.dot(x_ref[...], w3b_ref[...], **_DOT)
        + jnp.dot(x0_ref[...], w3c_ref[...], **_DOT)
        + jnp.log(dv)[:, None] * w3d_ref[...]
        + b31_ref[...]
    )
    h = jnp.maximum(pre, 0.0)
    out_ref[...] = jnp.dot(h, w32_ref[...], **_DOT) + b32_ref[...]


def _zero_rows(rows_v, n_rows, d):
    zero = jnp.zeros((LANES,), jnp.float32)

    def body(i, carry):
        for c in range(d // LANES):
            rows_v[i, pl.ds(c * LANES, LANES)] = zero
        return carry

    lax.fori_loop(0, n_rows, body, 0)


def _sc_segsum_body(n_chunks, acc_rows, table_hbm, gidx_hbm, sidx_hbm,
                    out_hbm, acc_sh, gidx, sidx, rows_v, sem):
    daug = table_hbm.shape[1]
    cid = lax.axis_index("c")
    tid = lax.axis_index("s")

    _zero_rows(rows_v, CH, daug)
    rows_per_tile = acc_rows // NS
    off = 0
    while off < rows_per_tile:
        step = min(CH, rows_per_tile - off)
        pltpu.sync_copy(rows_v.at[pl.ds(0, step)],
                        acc_sh.at[pl.ds(tid * rows_per_tile + off, step)])
        off += step
    plsc.subcore_barrier()

    wid = cid * NS + tid

    def chunk(g, carry):
        cidx = g * (NC * NS) + wid

        @pl.when(cidx < n_chunks)
        def _():
            pltpu.sync_copy(gidx_hbm.at[pl.ds(cidx, 1)], gidx)
            pltpu.sync_copy(sidx_hbm.at[pl.ds(cidx, 1)], sidx)
            pltpu.async_copy(table_hbm.at[gidx.at[0]], rows_v, sem).wait()
            pltpu.sync_copy(rows_v, acc_sh.at[sidx.at[0]], add=True)

        return carry

    n_rounds = -(-n_chunks // (NC * NS))
    lax.fori_loop(0, n_rounds, chunk, 0)
    plsc.subcore_barrier()

    pltpu.sync_copy(acc_sh.at[pl.ds(tid * rows_per_tile, rows_per_tile)],
                    out_hbm.at[cid, pl.ds(tid * rows_per_tile, rows_per_tile)])


def _sc_segsum(table, gidx2d, sidx2d, acc_rows):
    n_chunks = gidx2d.shape[0]
    daug = table.shape[1]
    mesh = plsc.VectorSubcoreMesh(core_axis_name="c", subcore_axis_name="s")
    return pl.kernel(
        functools.partial(_sc_segsum_body, n_chunks, acc_rows),
        out_type=jax.ShapeDtypeStruct((NC, acc_rows, daug), jnp.float32),
        mesh=mesh,
        scratch_types=[
            pltpu.VMEM_SHARED((acc_rows, daug), jnp.float32),
            pltpu.VMEM((1, CH), jnp.int32),
            pltpu.VMEM((1, CH), jnp.int32),
            pltpu.VMEM((CH, daug), jnp.float32),
            pltpu.SemaphoreType.DMA,
        ],
        compiler_params=pltpu.CompilerParams(use_tc_tiling_on_sc=False),
    )(table, gidx2d, sidx2d)


def kernel(X, vertex, edges, X0, W1_w, W1_b, W2_w, W2_b, W3_w1, W3_b1,
           W3_w2, W3_b2):
    n, d = X.shape
    e = vertex.shape[0]
    daug = d + LANES
    mp = 5120
    np_ = 10240
    assert d == 128 and e % CH == 0 and n <= np_
    n_chunks = e // CH

    f32 = jnp.float32
    zcol = jnp.zeros((d, LANES), f32)
    onescol = jnp.concatenate(
        [jnp.ones((1, 1), f32), jnp.zeros((1, LANES - 1), f32)], axis=1)

    w1p = jnp.concatenate([W1_w, zcol], axis=1)
    b1p = jnp.concatenate([W1_b.reshape(1, d), onescol], axis=1)
    w2a = W2_w[:d]
    w2bp = jnp.concatenate([W2_w[d:2 * d], zcol], axis=1)
    w2cp = jnp.concatenate([W2_w[2 * d:], jnp.zeros((1, LANES), f32)], axis=1)
    b2p = jnp.concatenate([W2_b.reshape(1, d), onescol], axis=1)
    w3a = W3_w1[:d]
    w3b = W3_w1[d:2 * d]
    w3c = W3_w1[2 * d:3 * d]
    w3d = W3_w1[3 * d:]

    vert2d = vertex.reshape(n_chunks, CH)
    edge2d = edges.reshape(n_chunks, CH)

    bn = 2000
    a_tab, bp_tab = pl.pallas_call(
        _tc_pre_body,
        grid=(n // bn,),
        in_specs=[
            pl.BlockSpec((bn, d), lambda i: (i, 0)),
            pl.BlockSpec((d, daug), lambda i: (0, 0)),
            pl.BlockSpec((1, daug), lambda i: (0, 0)),
            pl.BlockSpec((d, d), lambda i: (0, 0)),
        ],
        out_specs=[
            pl.BlockSpec((bn, daug), lambda i: (i, 0)),
            pl.BlockSpec((bn, d), lambda i: (i, 0)),
        ],
        out_shape=[
            jax.ShapeDtypeStruct((n, daug), f32),
            jax.ShapeDtypeStruct((n, d), f32),
        ],
    )(X, w1p, b1p, w2a)

    se_parts = _sc_segsum(a_tab, vert2d, edge2d, mp)

    bm = 512
    c_tab = pl.pallas_call(
        functools.partial(_tc_mid_body, d),
        grid=(mp // bm,),
        in_specs=[
            pl.BlockSpec((NC, bm, daug), lambda i: (0, i, 0)),
            pl.BlockSpec((d, daug), lambda i: (0, 0)),
            pl.BlockSpec((1, daug), lambda i: (0, 0)),
            pl.BlockSpec((1, daug), lambda i: (0, 0)),
        ],
        out_specs=pl.BlockSpec((bm, daug), lambda i: (i, 0)),
        out_shape=jax.ShapeDtypeStruct((mp, daug), f32),
    )(se_parts, w2bp, w2cp, b2p)

    t_parts = _sc_segsum(c_tab, edge2d, vert2d, np_)

    bp = 1280
    out = pl.pallas_call(
        functools.partial(_tc_post_body, d),
        grid=(np_ // bp,),
        in_specs=[
            pl.BlockSpec((NC, bp, daug), lambda i: (0, i, 0)),
            pl.BlockSpec((bp, d), lambda i: (i, 0)),
            pl.BlockSpec((bp, d), lambda i: (i, 0)),
            pl.BlockSpec((bp, d), lambda i: (i, 0)),
            pl.BlockSpec((d, d), lambda i: (0, 0)),
            pl.BlockSpec((d, d), lambda i: (0, 0)),
            pl.BlockSpec((d, d), lambda i: (0, 0)),
            pl.BlockSpec((1, d), lambda i: (0, 0)),
            pl.BlockSpec((1, d), lambda i: (0, 0)),
            pl.BlockSpec((d, d), lambda i: (0, 0)),
            pl.BlockSpec((1, d), lambda i: (0, 0)),
        ],
        out_specs=pl.BlockSpec((bp, d), lambda i: (i, 0)),
        out_shape=jax.ShapeDtypeStruct((n, d), f32),
    )(t_parts, bp_tab, X, X0, w3a, w3b, w3c, w3d,
      W3_b1.reshape(1, d), W3_w2, W3_b2.reshape(1, d))

    return out


# speedup vs baseline: 9.6726x; 1.6201x over previous
import functools

import jax
import jax.numpy as jnp
from jax import lax
from jax.experimental import pallas as pl
from jax.experimental.pallas import tpu as pltpu
from jax.experimental.pallas import tpu_sc as plsc

NC = 2
NS = 16
LANES = 16
CH = 128

_DOT = dict(preferred_element_type=jnp.float32, precision=lax.Precision.HIGHEST)


def _tc_pre_body(x_ref, w1p_ref, b1p_ref, w2a_ref, a_ref, bp_ref):
    x = x_ref[...]
    a_ref[...] = jnp.dot(x, w1p_ref[...], **_DOT) + b1p_ref[...]
    bp_ref[...] = jnp.dot(x, w2a_ref[...], **_DOT)


def _tc_mid_body(d, se_ref, w2bp_ref, w2cp_ref, b2p_ref, c_ref):
    se = se_ref[0] + se_ref[1]
    de = jnp.sum(se[:, d:], axis=1)
    xe = se[:, :d] / jnp.maximum(de, 1.0)[:, None]
    c_ref[...] = (
        jnp.dot(xe, w2bp_ref[...], **_DOT)
        + jnp.log(de)[:, None] * w2cp_ref[...]
        + b2p_ref[...]
    )


def _tc_post_body(d, t_ref, bp_ref, x_ref, x0_ref, w3a_ref, w3b_ref,
                  w3c_ref, w3d_ref, b31_ref, w32_ref, b32_ref, out_ref):
    t = t_ref[0] + t_ref[1]
    dv = jnp.sum(t[:, d:], axis=1)
    bp = bp_ref[...]
    xv = (dv[:, None] * bp + t[:, :d]) / jnp.maximum(dv, 1.0)[:, None]
    pre = (
        jnp.dot(xv, w3a_ref[...], **_DOT)
        + jnp.dot(x_ref[...], w3b_ref[...], **_DOT)
        + jnp.dot(x0_ref[...], w3c_ref[...], **_DOT)
        + jnp.log(dv)[:, None] * w3d_ref[...]
        + b31_ref[...]
    )
    h = jnp.maximum(pre, 0.0)
    out_ref[...] = jnp.dot(h, w32_ref[...], **_DOT) + b32_ref[...]


def _zero_rows(rows_v, n_rows, d):
    zero = jnp.zeros((LANES,), jnp.float32)

    def body(i, carry):
        for c in range(d // LANES):
            rows_v[i, pl.ds(c * LANES, LANES)] = zero
        return carry

    lax.fori_loop(0, n_rows, body, 0)


def _sc_segsum_body(n_chunks, acc_rows, table_hbm, gidx_hbm, sidx_hbm,
                    out_hbm, acc_sh, gidx, sidx, rows_v,
                    semi0, semi1, semg0, semg1):
    daug = table_hbm.shape[1]
    cid = lax.axis_index("c")
    tid = lax.axis_index("s")
    semi = (semi0, semi1)
    semg = (semg0, semg1)

    _zero_rows(rows_v.at[0], CH, daug)
    rows_per_tile = acc_rows // NS
    off = 0
    while off < rows_per_tile:
        step = min(CH, rows_per_tile - off)
        pltpu.sync_copy(rows_v.at[0].at[pl.ds(0, step)],
                        acc_sh.at[pl.ds(tid * rows_per_tile + off, step)])
        off += step
    plsc.subcore_barrier()

    wid = cid * NS + tid
    stride = NC * NS

    def idx_start(c, b):
        @pl.when(c < n_chunks)
        def _():
            pltpu.async_copy(gidx_hbm.at[pl.ds(c, 1)],
                             gidx.at[pl.ds(b, 1)], semi[b])
            pltpu.async_copy(sidx_hbm.at[pl.ds(c, 1)],
                             sidx.at[pl.ds(b, 1)], semi[b])

    def idx_wait(c, b):
        @pl.when(c < n_chunks)
        def _():
            pltpu.make_async_copy(gidx_hbm.at[pl.ds(c, 1)],
                                  gidx.at[pl.ds(b, 1)], semi[b]).wait()
            pltpu.make_async_copy(sidx_hbm.at[pl.ds(c, 1)],
                                  sidx.at[pl.ds(b, 1)], semi[b]).wait()

    def gather_start(c, b):
        @pl.when(c < n_chunks)
        def _():
            pltpu.async_copy(table_hbm.at[gidx.at[b]], rows_v.at[b], semg[b])

    def gather_wait(c, b):
        @pl.when(c < n_chunks)
        def _():
            pltpu.make_async_copy(table_hbm.at[gidx.at[b]],
                                  rows_v.at[b], semg[b]).wait()

    def scatter(c, b):
        @pl.when(c < n_chunks)
        def _():
            pltpu.sync_copy(rows_v.at[b], acc_sh.at[sidx.at[b]], add=True)

    idx_start(wid, 0)
    idx_start(stride + wid, 1)
    idx_wait(wid, 0)
    gather_start(wid, 0)

    n_rounds = -(-n_chunks // stride)

    def pair(g2, carry):
        for b in (0, 1):
            g = g2 * 2 + b
            c = g * stride + wid
            cn = c + stride
            idx_wait(cn, 1 - b)
            gather_wait(c, b)
            gather_start(cn, 1 - b)
            scatter(c, b)
            idx_start(cn + stride, b)
        return carry

    lax.fori_loop(0, -(-n_rounds // 2), pair, 0)
    plsc.subcore_barrier()

    pltpu.sync_copy(acc_sh.at[pl.ds(tid * rows_per_tile, rows_per_tile)],
                    out_hbm.at[cid, pl.ds(tid * rows_per_tile, rows_per_tile)])


def _sc_segsum(table, gidx2d, sidx2d, acc_rows):
    n_chunks = gidx2d.shape[0]
    daug = table.shape[1]
    mesh = plsc.VectorSubcoreMesh(core_axis_name="c", subcore_axis_name="s")
    return pl.kernel(
        functools.partial(_sc_segsum_body, n_chunks, acc_rows),
        out_type=jax.ShapeDtypeStruct((NC, acc_rows, daug), jnp.float32),
        mesh=mesh,
        scratch_types=[
            pltpu.VMEM_SHARED((acc_rows, daug), jnp.float32),
            pltpu.VMEM((2, CH), jnp.int32),
            pltpu.VMEM((2, CH), jnp.int32),
            pltpu.VMEM((2, CH, daug), jnp.float32),
            pltpu.SemaphoreType.DMA,
            pltpu.SemaphoreType.DMA,
            pltpu.SemaphoreType.DMA,
            pltpu.SemaphoreType.DMA,
        ],
        compiler_params=pltpu.CompilerParams(use_tc_tiling_on_sc=False),
    )(table, gidx2d, sidx2d)


def kernel(X, vertex, edges, X0, W1_w, W1_b, W2_w, W2_b, W3_w1, W3_b1,
           W3_w2, W3_b2):
    n, d = X.shape
    e = vertex.shape[0]
    daug = d + LANES
    mp = 5120
    np_ = 10240
    assert d == 128 and e % CH == 0 and n <= np_
    n_chunks = e // CH

    f32 = jnp.float32
    zcol = jnp.zeros((d, LANES), f32)
    onescol = jnp.concatenate(
        [jnp.ones((1, 1), f32), jnp.zeros((1, LANES - 1), f32)], axis=1)

    w1p = jnp.concatenate([W1_w, zcol], axis=1)
    b1p = jnp.concatenate([W1_b.reshape(1, d), onescol], axis=1)
    w2a = W2_w[:d]
    w2bp = jnp.concatenate([W2_w[d:2 * d], zcol], axis=1)
    w2cp = jnp.concatenate([W2_w[2 * d:], jnp.zeros((1, LANES), f32)], axis=1)
    b2p = jnp.concatenate([W2_b.reshape(1, d), onescol], axis=1)
    w3a = W3_w1[:d]
    w3b = W3_w1[d:2 * d]
    w3c = W3_w1[2 * d:3 * d]
    w3d = W3_w1[3 * d:]

    vert2d = vertex.reshape(n_chunks, CH)
    edge2d = edges.reshape(n_chunks, CH)

    bn = 2000
    a_tab, bp_tab = pl.pallas_call(
        _tc_pre_body,
        grid=(n // bn,),
        in_specs=[
            pl.BlockSpec((bn, d), lambda i: (i, 0)),
            pl.BlockSpec((d, daug), lambda i: (0, 0)),
            pl.BlockSpec((1, daug), lambda i: (0, 0)),
            pl.BlockSpec((d, d), lambda i: (0, 0)),
        ],
        out_specs=[
            pl.BlockSpec((bn, daug), lambda i: (i, 0)),
            pl.BlockSpec((bn, d), lambda i: (i, 0)),
        ],
        out_shape=[
            jax.ShapeDtypeStruct((n, daug), f32),
            jax.ShapeDtypeStruct((n, d), f32),
        ],
    )(X, w1p, b1p, w2a)

    se_parts = _sc_segsum(a_tab, vert2d, edge2d, mp)

    bm = 512
    c_tab = pl.pallas_call(
        functools.partial(_tc_mid_body, d),
        grid=(mp // bm,),
        in_specs=[
            pl.BlockSpec((NC, bm, daug), lambda i: (0, i, 0)),
            pl.BlockSpec((d, daug), lambda i: (0, 0)),
            pl.BlockSpec((1, daug), lambda i: (0, 0)),
            pl.BlockSpec((1, daug), lambda i: (0, 0)),
        ],
        out_specs=pl.BlockSpec((bm, daug), lambda i: (i, 0)),
        out_shape=jax.ShapeDtypeStruct((mp, daug), f32),
    )(se_parts, w2bp, w2cp, b2p)

    t_parts = _sc_segsum(c_tab, edge2d, vert2d, np_)

    bp = 1280
    out = pl.pallas_call(
        functools.partial(_tc_post_body, d),
        grid=(np_ // bp,),
        in_specs=[
            pl.BlockSpec((NC, bp, daug), lambda i: (0, i, 0)),
            pl.BlockSpec((bp, d), lambda i: (i, 0)),
            pl.BlockSpec((bp, d), lambda i: (i, 0)),
            pl.BlockSpec((bp, d), lambda i: (i, 0)),
            pl.BlockSpec((d, d), lambda i: (0, 0)),
            pl.BlockSpec((d, d), lambda i: (0, 0)),
            pl.BlockSpec((d, d), lambda i: (0, 0)),
            pl.BlockSpec((1, d), lambda i: (0, 0)),
            pl.BlockSpec((1, d), lambda i: (0, 0)),
            pl.BlockSpec((d, d), lambda i: (0, 0)),
            pl.BlockSpec((1, d), lambda i: (0, 0)),
        ],
        out_specs=pl.BlockSpec((bp, d), lambda i: (i, 0)),
        out_shape=jax.ShapeDtypeStruct((n, d), f32),
    )(t_parts, bp_tab, X, X0, w3a, w3b, w3c, w3d,
      W3_b1.reshape(1, d), W3_w2, W3_b2.reshape(1, d))

    return out
